# Initial kernel scaffold; baseline (speedup 1.0000x reference)
import jax
import jax.numpy as jnp
from jax.experimental import pallas as pl


def kernel(x, edge_index, W0, a_src0, a_dst0, b0, W1, a_src1, a_dst1, b1, W2, a_src2, a_dst2, b2, W_lin, b_lin):
    raise NotImplementedError("write your pallas kernel here")


# speedup vs baseline: 8.2669x; 8.2669x over previous
import functools
import jax
import jax.numpy as jnp
from jax import lax
from jax.experimental import pallas as pl
from jax.experimental.pallas import tpu as pltpu
import jax.experimental.pallas.tpu_sc as plsc

N = 10000
NPAD = 10240
E = 320000
H = 8
C = 256
OUT = 10

CHUNK = 128
NCHUNK = E // CHUNK
NC, NS = 2, 16
NW = NC * NS
RPS = NPAD // NS


def _proj_body(ndc, x_ref, w_ref, as_ref, ad_ref, ts_ref, td_ref):
    xs = jnp.concatenate([x_ref[i] for i in range(ndc)], axis=1)
    w = w_ref[...]
    ws = jnp.sum(w * as_ref[...][None], axis=2)
    wd = jnp.sum(w * ad_ref[...][None], axis=2)
    es = jnp.dot(xs, ws, preferred_element_type=jnp.float32)
    ed = jnp.dot(xs, wd, preferred_element_type=jnp.float32)
    ts_ref[...] = jnp.concatenate([es, es], axis=1)
    td_ref[...] = jnp.concatenate([ed, ed], axis=1)


def _proj(x4, W, a_s, a_d):
    ndc = x4.shape[0]
    return pl.pallas_call(
        functools.partial(_proj_body, ndc),
        out_shape=[jax.ShapeDtypeStruct((NPAD, 2 * H), jnp.float32),
                   jax.ShapeDtypeStruct((NPAD, 2 * H), jnp.float32)],
    )(x4, W, a_s, a_d)


def _edge_body(ts_hbm, td_hbm, src_hbm, dst_hbm, zz_hbm, ex_hbm, sp_hbm,
               idx_s3, idx_d3, idx_s, idx_d, rows_s, rows_d, ex_buf,
               sem1, sem2, s_acc):
    core = lax.axis_index("c")
    sub = lax.axis_index("s")
    wid = sub * NC + core

    r0 = sub * RPS
    pltpu.sync_copy(zz_hbm.at[pl.ds(r0, RPS)], s_acc.at[pl.ds(r0, RPS)])
    plsc.subcore_barrier()

    def chunk_step(k, _):
        c = k * NW + wid

        @pl.when(c < NCHUNK)
        def _():
            pltpu.sync_copy(src_hbm.at[c], idx_s3)
            pltpu.sync_copy(dst_hbm.at[c], idx_d3)
            for q in range(CHUNK // 16):
                sl = pl.ds(q * 16, 16)
                idx_s[sl] = idx_s3[0, sl]
                idx_d[sl] = idx_d3[0, sl]
            pltpu.async_copy(ts_hbm.at[idx_s], rows_s, sem1).wait()
            pltpu.async_copy(td_hbm.at[idx_d], rows_d, sem2).wait()

            def row_step(i, _):
                v = rows_s[i] + rows_d[i]
                l = jnp.where(v >= 0.0, v, 0.2 * v)
                ex_buf[i] = jnp.exp(l)
                return 0

            lax.fori_loop(0, CHUNK, row_step, 0)
            pltpu.sync_copy(ex_buf, ex_hbm.at[c])
            pltpu.sync_copy(ex_buf, s_acc.at[idx_d], add=True)
        return 0

    lax.fori_loop(0, (NCHUNK + NW - 1) // NW, chunk_step, 0)
    plsc.subcore_barrier()
    pltpu.sync_copy(s_acc.at[pl.ds(r0, RPS)],
                    sp_hbm.at[core, pl.ds(r0, RPS)])


def _edge_phase(t_s, t_d, src3d, dst3d, zz16):
    mesh = plsc.VectorSubcoreMesh(core_axis_name="c", subcore_axis_name="s")
    return pl.kernel(
        _edge_body,
        compiler_params=pltpu.CompilerParams(use_tc_tiling_on_sc=False),
        out_type=[jax.ShapeDtypeStruct((NCHUNK, CHUNK, 2 * H), jnp.float32),
                  jax.ShapeDtypeStruct((NC, NPAD, 2 * H), jnp.float32)],
        mesh=mesh,
        scratch_types=[
            pltpu.VMEM((1, CHUNK), jnp.int32),
            pltpu.VMEM((1, CHUNK), jnp.int32),
            pltpu.VMEM((CHUNK,), jnp.int32),
            pltpu.VMEM((CHUNK,), jnp.int32),
            pltpu.VMEM((CHUNK, 2 * H), jnp.float32),
            pltpu.VMEM((CHUNK, 2 * H), jnp.float32),
            pltpu.VMEM((CHUNK, 2 * H), jnp.float32),
            pltpu.SemaphoreType.DMA,
            pltpu.SemaphoreType.DMA,
            pltpu.VMEM_SHARED((NPAD, 2 * H), jnp.float32),
        ],
    )(t_s, t_d, src3d, dst3d, zz16)


def _agg_body(ndc, xflat_hbm, src_hbm, dst_hbm, ex_hbm, zz_hbm, u_hbm,
              idx_s3, idx_d3, idx_s, idx_d, xbuf, ex_buf, srow, sem1, u_acc):
    pairs_per_sc = (H * ndc) // NC
    core = lax.axis_index("c")
    sub = lax.axis_index("s")
    r0 = sub * RPS
    dnums = lax.GatherDimensionNumbers(
        offset_dims=(), collapsed_slice_dims=(0,), start_index_map=(0,))

    def pair_step(p_idx, _):
        pid = core * pairs_per_sc + p_idx
        h = pid // ndc
        dc = pid % ndc
        hsplat = jnp.zeros((16,), jnp.int32) + h

        pltpu.sync_copy(zz_hbm.at[pl.ds(r0, RPS)], u_acc.at[pl.ds(r0, RPS)])
        plsc.subcore_barrier()

        def chunk_step(k, _):
            c = k * NS + sub

            @pl.when(c < NCHUNK)
            def _():
                pltpu.sync_copy(src_hbm.at[c], idx_s3)
                pltpu.sync_copy(dst_hbm.at[c], idx_d3)
                off = dc * NPAD
                for q in range(CHUNK // 16):
                    sl = pl.ds(q * 16, 16)
                    idx_s[sl] = idx_s3[0, sl] + off
                    idx_d[sl] = idx_d3[0, sl]
                pltpu.async_copy(xflat_hbm.at[idx_s], xbuf, sem1).wait()
                pltpu.sync_copy(ex_hbm.at[c], ex_buf)

                def row_step(i, _):
                    a = lax.gather(
                        ex_buf[i], hsplat[:, None], dnums, (1,),
                        mode=lax.GatherScatterMode.PROMISE_IN_BOUNDS)
                    for q in range(8):
                        sl = pl.ds(q * 16, 16)
                        srow[i, sl] = xbuf[i, sl] * a
                    return 0

                lax.fori_loop(0, CHUNK, row_step, 0)
                pltpu.sync_copy(srow, u_acc.at[idx_d], add=True)
            return 0

        lax.fori_loop(0, (NCHUNK + NS - 1) // NS, chunk_step, 0)
        plsc.subcore_barrier()
        pltpu.sync_copy(u_acc.at[pl.ds(r0, RPS)],
                        u_hbm.at[pid, pl.ds(r0, RPS)])
        return 0

    lax.fori_loop(0, pairs_per_sc, pair_step, 0)


def _aggregate(xflat, src3d, dst3d, ex_e, zz128, ndc):
    mesh = plsc.VectorSubcoreMesh(core_axis_name="c", subcore_axis_name="s")
    return pl.kernel(
        functools.partial(_agg_body, ndc),
        out_type=jax.ShapeDtypeStruct((H * ndc, NPAD, 128), jnp.float32),
        mesh=mesh,
        compiler_params=pltpu.CompilerParams(use_tc_tiling_on_sc=False),
        scratch_types=[
            pltpu.VMEM((1, CHUNK), jnp.int32),
            pltpu.VMEM((1, CHUNK), jnp.int32),
            pltpu.VMEM((CHUNK,), jnp.int32),
            pltpu.VMEM((CHUNK,), jnp.int32),
            pltpu.VMEM((CHUNK, 128), jnp.float32),
            pltpu.VMEM((CHUNK, 2 * H), jnp.float32),
            pltpu.VMEM((CHUNK, 128), jnp.float32),
            pltpu.SemaphoreType.DMA,
            pltpu.VMEM_SHARED((NPAD, 128), jnp.float32),
        ],
    )(xflat, src3d, dst3d, ex_e, zz128)


def _comb_body(ndc, ndc_out, final, u_ref, sp_ref, wt_ref, b_ref,
               wlin_ref, blin_ref, o_ref):
    pairs = H * ndc
    s = sp_ref[0, :, :H] + sp_ref[1, :, :H] + 1e-16
    acc = jnp.zeros((u_ref.shape[1], C), jnp.float32)
    for pid in range(pairs):
        h = pid // ndc
        r = u_ref[pid] / s[:, h][:, None]
        acc = acc + jnp.dot(r, wt_ref[pid],
                            preferred_element_type=jnp.float32)
    y = jax.nn.relu(acc * (1.0 / H) + b_ref[...][None])
    if final:
        o_ref[...] = jnp.dot(y, wlin_ref[...],
                             preferred_element_type=jnp.float32) \
            + blin_ref[...][None]
    else:
        for j in range(ndc_out):
            o_ref[j] = y[:, j * 128:(j + 1) * 128]


def _combine(u, s_part, W_t, b, W_lin, b_lin, ndc, final):
    bn = 1024
    pairs = H * ndc
    ndc_out = C // 128
    if final:
        out_shape = jax.ShapeDtypeStruct((NPAD, OUT), jnp.float32)
        out_spec = pl.BlockSpec((bn, OUT), lambda i: (i, 0))
    else:
        out_shape = jax.ShapeDtypeStruct((ndc_out, NPAD, 128), jnp.float32)
        out_spec = pl.BlockSpec((ndc_out, bn, 128), lambda i: (0, i, 0))
    return pl.pallas_call(
        functools.partial(_comb_body, ndc, ndc_out, final),
        grid=(NPAD // bn,),
        in_specs=[
            pl.BlockSpec((pairs, bn, 128), lambda i: (0, i, 0)),
            pl.BlockSpec((NC, bn, 2 * H), lambda i: (0, i, 0)),
            pl.BlockSpec((pairs, 128, C), lambda i: (0, 0, 0)),
            pl.BlockSpec((C,), lambda i: (0,)),
            pl.BlockSpec((C, OUT), lambda i: (0, 0)),
            pl.BlockSpec((OUT,), lambda i: (0,)),
        ],
        out_specs=out_spec,
        out_shape=out_shape,
    )(u, s_part, W_t, b, W_lin, b_lin)


def kernel(x, edge_index, W0, a_src0, a_dst0, b0, W1, a_src1, a_dst1, b1,
           W2, a_src2, a_dst2, b2, W_lin, b_lin):
    src3d = edge_index[0].astype(jnp.int32).reshape(NCHUNK, 1, CHUNK)
    dst3d = edge_index[1].astype(jnp.int32).reshape(NCHUNK, 1, CHUNK)
    zz16 = jnp.zeros((NPAD, 2 * H), jnp.float32)
    zz128 = jnp.zeros((NPAD, 128), jnp.float32)

    x_cur = jnp.pad(x, ((0, NPAD - N), (0, 0))).reshape(1, NPAD, 128)
    layers = ((W0, a_src0, a_dst0, b0), (W1, a_src1, a_dst1, b1),
              (W2, a_src2, a_dst2, b2))
    for l, (W, a_s, a_d, b) in enumerate(layers):
        ndc = x_cur.shape[0]
        t_s, t_d = _proj(x_cur, W, a_s, a_d)
        ex_e, s_part = _edge_phase(t_s, t_d, src3d, dst3d, zz16)
        xflat = x_cur.reshape(ndc * NPAD, 128)
        u = _aggregate(xflat, src3d, dst3d, ex_e, zz128, ndc)
        W_t = W.reshape(ndc, 128, H, C).transpose(2, 0, 1, 3) \
            .reshape(H * ndc, 128, C)
        x_cur = _combine(u, s_part, W_t, b, W_lin, b_lin, ndc,
                         final=(l == 2))
    return x_cur[:N]


# speedup vs baseline: 19.6409x; 2.3758x over previous
import functools
import jax
import jax.numpy as jnp
from jax import lax
from jax.experimental import pallas as pl
from jax.experimental.pallas import tpu as pltpu
import jax.experimental.pallas.tpu_sc as plsc

N = 10000
NPAD = 10240
E = 320000
H = 8
C = 256
OUT = 10

CHUNK = 128
NCHUNK = E // CHUNK
NC, NS = 2, 16
NW = NC * NS
RPS = NPAD // NS


def _proj_body(ndc, x_ref, w_ref, as_ref, ad_ref, ts_ref, td_ref):
    xs = jnp.concatenate([x_ref[i] for i in range(ndc)], axis=1)
    w = w_ref[...]
    ws = jnp.sum(w * as_ref[...][None], axis=2)
    wd = jnp.sum(w * ad_ref[...][None], axis=2)
    es = jnp.dot(xs, ws, preferred_element_type=jnp.float32)
    ed = jnp.dot(xs, wd, preferred_element_type=jnp.float32)
    ts_ref[...] = jnp.concatenate([es, es], axis=1)
    td_ref[...] = jnp.concatenate([ed, ed], axis=1)


def _proj(x4, W, a_s, a_d):
    ndc = x4.shape[0]
    return pl.pallas_call(
        functools.partial(_proj_body, ndc),
        out_shape=[jax.ShapeDtypeStruct((NPAD, 2 * H), jnp.float32),
                   jax.ShapeDtypeStruct((NPAD, 2 * H), jnp.float32)],
    )(x4, W, a_s, a_d)


def _edge_body(ts_hbm, td_hbm, src_hbm, dst_hbm, zz_hbm, ex_hbm, sp_hbm,
               idx_s3, idx_d3, idx_s, idx_d, rows_s, rows_d, ex_buf,
               sem1, sem2, s_acc):
    core = lax.axis_index("c")
    sub = lax.axis_index("s")
    wid = sub * NC + core

    r0 = sub * RPS
    pltpu.sync_copy(zz_hbm.at[pl.ds(r0, RPS)], s_acc.at[pl.ds(r0, RPS)])
    plsc.subcore_barrier()

    def chunk_step(k, _):
        c = k * NW + wid

        @pl.when(c < NCHUNK)
        def _():
            pltpu.sync_copy(src_hbm.at[c], idx_s3)
            pltpu.sync_copy(dst_hbm.at[c], idx_d3)
            for q in range(CHUNK // 16):
                sl = pl.ds(q * 16, 16)
                idx_s[sl] = idx_s3[0, sl]
                idx_d[sl] = idx_d3[0, sl]
            pltpu.async_copy(ts_hbm.at[idx_s], rows_s, sem1).wait()
            pltpu.async_copy(td_hbm.at[idx_d], rows_d, sem2).wait()

            def row_step(i, _):
                v = rows_s[i] + rows_d[i]
                l = jnp.where(v >= 0.0, v, 0.2 * v)
                ex_buf[i] = jnp.exp(l)
                return 0

            lax.fori_loop(0, CHUNK, row_step, 0)
            pltpu.sync_copy(ex_buf, ex_hbm.at[c])
            pltpu.sync_copy(ex_buf, s_acc.at[idx_d], add=True)
        return 0

    lax.fori_loop(0, (NCHUNK + NW - 1) // NW, chunk_step, 0)
    plsc.subcore_barrier()
    pltpu.sync_copy(s_acc.at[pl.ds(r0, RPS)],
                    sp_hbm.at[core, pl.ds(r0, RPS)])


def _edge_phase(t_s, t_d, src3d, dst3d, zz16):
    mesh = plsc.VectorSubcoreMesh(core_axis_name="c", subcore_axis_name="s")
    return pl.kernel(
        _edge_body,
        compiler_params=pltpu.CompilerParams(use_tc_tiling_on_sc=False),
        out_type=[jax.ShapeDtypeStruct((NCHUNK, CHUNK, 2 * H), jnp.float32),
                  jax.ShapeDtypeStruct((NC, NPAD, 2 * H), jnp.float32)],
        mesh=mesh,
        scratch_types=[
            pltpu.VMEM((1, CHUNK), jnp.int32),
            pltpu.VMEM((1, CHUNK), jnp.int32),
            pltpu.VMEM((CHUNK,), jnp.int32),
            pltpu.VMEM((CHUNK,), jnp.int32),
            pltpu.VMEM((CHUNK, 2 * H), jnp.float32),
            pltpu.VMEM((CHUNK, 2 * H), jnp.float32),
            pltpu.VMEM((CHUNK, 2 * H), jnp.float32),
            pltpu.SemaphoreType.DMA,
            pltpu.SemaphoreType.DMA,
            pltpu.VMEM_SHARED((NPAD, 2 * H), jnp.float32),
        ],
    )(t_s, t_d, src3d, dst3d, zz16)


CH = 64
NCHC = E // CH
CPSC = (NCHC + NS - 1) // NS
HC = 80
NQ = 4
NCHCP = NS * CPSC + NQ * HC


def _agg_body(ndc, xflat_hbm, src_hbm, dst_hbm, ex_hbm, zz_hbm, u_hbm,
              ias, iad, is_a, is_b, id_a, id_b, x_a, x_b, ex_a, ex_b,
              sr_a, sr_b, sg_a, sg_b, sx_a, sx_b, sc_a, sc_b, u_acc):
    pairs_per_sc = (H * ndc) // NC
    core = lax.axis_index("c")
    sub = lax.axis_index("s")
    r0 = sub * RPS
    c0 = sub * CPSC
    nch = jnp.minimum(CPSC, NCHC - c0)
    dnums = lax.GatherDimensionNumbers(
        offset_dims=(), collapsed_slice_dims=(0,), start_index_map=(0,))

    def wr_idx(dref, sref, c, off):
        for q in range(CH // 16):
            sl = pl.ds(q * 16, 16)
            dref[sl] = sref[c, 0, sl] + off

    def pair_step(p_idx, _):
        pid = core * pairs_per_sc + p_idx
        h = pid // ndc
        dc = pid % ndc
        off = dc * NPAD
        hsplat = jnp.zeros((16,), jnp.int32) + h

        pltpu.sync_copy(zz_hbm.at[pl.ds(r0, RPS)], u_acc.at[pl.ds(r0, RPS)])
        plsc.subcore_barrier()

        for half in range(NQ):
            h0 = half * HC
            lim = jnp.clip(nch - h0, 0, HC)
            pltpu.sync_copy(src_hbm.at[pl.ds(c0 + h0, HC)], ias)
            pltpu.sync_copy(dst_hbm.at[pl.ds(c0 + h0, HC)], iad)

            wr_idx(is_a, ias, 0, off)
            pltpu.async_copy(xflat_hbm.at[is_a], x_a, sg_a)
            pltpu.async_copy(ex_hbm.at[c0 + h0], ex_a, sx_a)
            wr_idx(is_b, ias, 1, off)
            pltpu.async_copy(xflat_hbm.at[is_b], x_b, sg_b)
            pltpu.async_copy(ex_hbm.at[c0 + h0 + 1], ex_b, sx_b)

            def body(j, _):
                for par, is_x, id_x, x_x, ex_x, sr_x, sg_x, sx_x, sc_x in (
                        (0, is_a, id_a, x_a, ex_a, sr_a, sg_a, sx_a, sc_a),
                        (1, is_b, id_b, x_b, ex_b, sr_b, sg_b, sx_b, sc_b)):
                    lc = 2 * j + par

                    @pl.when(lc < lim)
                    def _():
                        pltpu.make_async_copy(
                            xflat_hbm.at[pl.ds(0, CH)], x_x, sg_x).wait()
                        pltpu.make_async_copy(
                            ex_hbm.at[0], ex_x, sx_x).wait()

                        @pl.when(j > 0)
                        def _():
                            pltpu.make_async_copy(
                                xflat_hbm.at[pl.ds(0, CH)], sr_x,
                                sc_x).wait()

                        def row_step(i, _):
                            a = lax.gather(
                                ex_x[i], hsplat[:, None], dnums, (1,),
                                mode=lax.GatherScatterMode.PROMISE_IN_BOUNDS)
                            for q in range(8):
                                sl = pl.ds(q * 16, 16)
                                sr_x[i, sl] = x_x[i, sl] * a
                            return 0

                        lax.fori_loop(0, CH, row_step, 0)
                        wr_idx(id_x, iad, lc, 0)
                        pltpu.async_copy(sr_x, u_acc.at[id_x], sc_x,
                                         add=True)

                        @pl.when(lc + 2 < lim)
                        def _():
                            wr_idx(is_x, ias, lc + 2, off)
                            pltpu.async_copy(xflat_hbm.at[is_x], x_x, sg_x)
                            pltpu.async_copy(ex_hbm.at[c0 + h0 + lc + 2],
                                             ex_x, sx_x)
                return 0

            lax.fori_loop(0, HC // 2, body, 0)
            pltpu.make_async_copy(xflat_hbm.at[pl.ds(0, CH)], sr_a,
                                  sc_a).wait()
            pltpu.make_async_copy(xflat_hbm.at[pl.ds(0, CH)], sr_b,
                                  sc_b).wait()
        plsc.subcore_barrier()
        pltpu.sync_copy(u_acc.at[pl.ds(r0, RPS)],
                        u_hbm.at[pid, pl.ds(r0, RPS)])
        return 0

    lax.fori_loop(0, pairs_per_sc, pair_step, 0)


def _aggregate(xflat, src16c, dst16c, ex_c, zz128, ndc):
    mesh = plsc.VectorSubcoreMesh(core_axis_name="c", subcore_axis_name="s")
    return pl.kernel(
        functools.partial(_agg_body, ndc),
        out_type=jax.ShapeDtypeStruct((H * ndc, NPAD, 128), jnp.float32),
        mesh=mesh,
        compiler_params=pltpu.CompilerParams(use_tc_tiling_on_sc=False),
        scratch_types=[
            pltpu.VMEM((HC, 1, CH), jnp.int32),
            pltpu.VMEM((HC, 1, CH), jnp.int32),
            pltpu.VMEM((CH,), jnp.int32),
            pltpu.VMEM((CH,), jnp.int32),
            pltpu.VMEM((CH,), jnp.int32),
            pltpu.VMEM((CH,), jnp.int32),
            pltpu.VMEM((CH, 128), jnp.float32),
            pltpu.VMEM((CH, 128), jnp.float32),
            pltpu.VMEM((CH, 2 * H), jnp.float32),
            pltpu.VMEM((CH, 2 * H), jnp.float32),
            pltpu.VMEM((CH, 128), jnp.float32),
            pltpu.VMEM((CH, 128), jnp.float32),
            pltpu.SemaphoreType.DMA,
            pltpu.SemaphoreType.DMA,
            pltpu.SemaphoreType.DMA,
            pltpu.SemaphoreType.DMA,
            pltpu.SemaphoreType.DMA,
            pltpu.SemaphoreType.DMA,
            pltpu.VMEM_SHARED((NPAD, 128), jnp.float32),
        ],
    )(xflat, src16c, dst16c, ex_c, zz128)


def _comb_body(ndc, ndc_out, final, u_ref, sp_ref, wt_ref, b_ref,
               wlin_ref, blin_ref, o_ref):
    pairs = H * ndc
    s = sp_ref[0, :, :H] + sp_ref[1, :, :H] + 1e-16
    acc = jnp.zeros((u_ref.shape[1], C), jnp.float32)
    for pid in range(pairs):
        h = pid // ndc
        r = u_ref[pid] / s[:, h][:, None]
        acc = acc + jnp.dot(r, wt_ref[pid],
                            preferred_element_type=jnp.float32)
    y = jax.nn.relu(acc * (1.0 / H) + b_ref[...][None])
    if final:
        o_ref[...] = jnp.dot(y, wlin_ref[...],
                             preferred_element_type=jnp.float32) \
            + blin_ref[...][None]
    else:
        for j in range(ndc_out):
            o_ref[j] = y[:, j * 128:(j + 1) * 128]


def _combine(u, s_part, W_t, b, W_lin, b_lin, ndc, final):
    bn = 1024
    pairs = H * ndc
    ndc_out = C // 128
    if final:
        out_shape = jax.ShapeDtypeStruct((NPAD, OUT), jnp.float32)
        out_spec = pl.BlockSpec((bn, OUT), lambda i: (i, 0))
    else:
        out_shape = jax.ShapeDtypeStruct((ndc_out, NPAD, 128), jnp.float32)
        out_spec = pl.BlockSpec((ndc_out, bn, 128), lambda i: (0, i, 0))
    return pl.pallas_call(
        functools.partial(_comb_body, ndc, ndc_out, final),
        grid=(NPAD // bn,),
        in_specs=[
            pl.BlockSpec((pairs, bn, 128), lambda i: (0, i, 0)),
            pl.BlockSpec((NC, bn, 2 * H), lambda i: (0, i, 0)),
            pl.BlockSpec((pairs, 128, C), lambda i: (0, 0, 0)),
            pl.BlockSpec((C,), lambda i: (0,)),
            pl.BlockSpec((C, OUT), lambda i: (0, 0)),
            pl.BlockSpec((OUT,), lambda i: (0,)),
        ],
        out_specs=out_spec,
        out_shape=out_shape,
    )(u, s_part, W_t, b, W_lin, b_lin)


def kernel(x, edge_index, W0, a_src0, a_dst0, b0, W1, a_src1, a_dst1, b1,
           W2, a_src2, a_dst2, b2, W_lin, b_lin):
    src32 = edge_index[0].astype(jnp.int32)
    dst32 = edge_index[1].astype(jnp.int32)
    src16b = src32.reshape(NCHUNK, 1, CHUNK)
    dst16b = dst32.reshape(NCHUNK, 1, CHUNK)
    src16c = jnp.pad(src32.reshape(NCHC, 1, CH),
                     ((0, NCHCP - NCHC), (0, 0), (0, 0)))
    dst16c = jnp.pad(dst32.reshape(NCHC, 1, CH),
                     ((0, NCHCP - NCHC), (0, 0), (0, 0)))
    zz16 = jnp.zeros((NPAD, 2 * H), jnp.float32)
    zz128 = jnp.zeros((NPAD, 128), jnp.float32)

    x_cur = jnp.pad(x, ((0, NPAD - N), (0, 0))).reshape(1, NPAD, 128)
    layers = ((W0, a_src0, a_dst0, b0), (W1, a_src1, a_dst1, b1),
              (W2, a_src2, a_dst2, b2))
    for l, (W, a_s, a_d, b) in enumerate(layers):
        ndc = x_cur.shape[0]
        t_s, t_d = _proj(x_cur, W, a_s, a_d)
        ex_e, s_part = _edge_phase(t_s, t_d, src16b, dst16b, zz16)
        xflat = x_cur.reshape(ndc * NPAD, 128)
        ex_c = ex_e.reshape(NCHC, CH, 2 * H)
        u = _aggregate(xflat, src16c, dst16c, ex_c, zz128, ndc)
        W_t = W.reshape(ndc, 128, H, C).transpose(2, 0, 1, 3) \
            .reshape(H * ndc, 128, C)
        x_cur = _combine(u, s_part, W_t, b, W_lin, b_lin, ndc,
                         final=(l == 2))
    return x_cur[:N]


# speedup vs baseline: 20.8027x; 1.0592x over previous
import functools
import jax
import jax.numpy as jnp
from jax import lax
from jax.experimental import pallas as pl
from jax.experimental.pallas import tpu as pltpu
import jax.experimental.pallas.tpu_sc as plsc

N = 10000
NPAD = 10240
E = 320000
H = 8
C = 256
OUT = 10

CHUNK = 128
NCHUNK = E // CHUNK
NC, NS = 2, 16
NW = NC * NS
RPS = NPAD // NS


def _proj_body(ndc, x_ref, w_ref, as_ref, ad_ref, ts_ref, td_ref):
    xs = jnp.concatenate([x_ref[i] for i in range(ndc)], axis=1)
    w = w_ref[...]
    ws = jnp.sum(w * as_ref[...][None], axis=2)
    wd = jnp.sum(w * ad_ref[...][None], axis=2)
    es = jnp.dot(xs, ws, preferred_element_type=jnp.float32)
    ed = jnp.dot(xs, wd, preferred_element_type=jnp.float32)
    ts_ref[...] = jnp.concatenate([es, es], axis=1)
    td_ref[...] = jnp.concatenate([ed, ed], axis=1)


def _proj(x4, W, a_s, a_d):
    ndc = x4.shape[0]
    return pl.pallas_call(
        functools.partial(_proj_body, ndc),
        out_shape=[jax.ShapeDtypeStruct((NPAD, 2 * H), jnp.float32),
                   jax.ShapeDtypeStruct((NPAD, 2 * H), jnp.float32)],
    )(x4, W, a_s, a_d)


CPSB = (NCHUNK + NS - 1) // NS
NCHBP = NS * CPSB


def _edge_body(ts_hbm, td_hbm, src_hbm, dst_hbm, zz_hbm, ex_hbm, sp_hbm,
               ias, iad, is_a, is_b, ig_a, ig_b, id_a, id_b,
               rs_a, rs_b, rd_a, rd_b, ex_a, ex_b,
               ss_a, ss_b, sd_a, sd_b, sw_a, sw_b, sc_a, sc_b, s_acc):
    core = lax.axis_index("c")
    sub = lax.axis_index("s")
    r0 = sub * RPS
    c0 = sub * CPSB
    nch = jnp.minimum(CPSB, NCHUNK - c0)

    pltpu.sync_copy(zz_hbm.at[pl.ds(r0, RPS)], s_acc.at[pl.ds(r0, RPS)])
    pltpu.sync_copy(src_hbm.at[pl.ds(c0, CPSB)], ias)
    pltpu.sync_copy(dst_hbm.at[pl.ds(c0, CPSB)], iad)
    plsc.subcore_barrier()

    def wr_idx(dref, sref, c):
        for q in range(CHUNK // 16):
            sl = pl.ds(q * 16, 16)
            dref[sl] = sref[c, 0, sl]

    wr_idx(is_a, ias, 0)
    wr_idx(ig_a, iad, 0)
    pltpu.async_copy(ts_hbm.at[is_a], rs_a, ss_a)
    pltpu.async_copy(td_hbm.at[ig_a], rd_a, sd_a)
    wr_idx(is_b, ias, 1)
    wr_idx(ig_b, iad, 1)
    pltpu.async_copy(ts_hbm.at[is_b], rs_b, ss_b)
    pltpu.async_copy(td_hbm.at[ig_b], rd_b, sd_b)

    def body(j, _):
        for par, is_x, ig_x, id_x, rs_x, rd_x, ex_x, ss_x, sd_x, sw_x, \
                sc_x in (
                (0, is_a, ig_a, id_a, rs_a, rd_a, ex_a, ss_a, sd_a, sw_a,
                 sc_a),
                (1, is_b, ig_b, id_b, rs_b, rd_b, ex_b, ss_b, sd_b, sw_b,
                 sc_b)):
            c = 2 * j + par

            @pl.when(c < nch)
            def _():
                pltpu.make_async_copy(ts_hbm.at[pl.ds(0, CHUNK)], rs_x,
                                      ss_x).wait()
                pltpu.make_async_copy(td_hbm.at[pl.ds(0, CHUNK)], rd_x,
                                      sd_x).wait()

                @pl.when(j > 0)
                def _():
                    pltpu.make_async_copy(ex_x, ex_hbm.at[0], sw_x).wait()
                    pltpu.make_async_copy(ts_hbm.at[pl.ds(0, CHUNK)],
                                          ex_x, sc_x).wait()

                def row_step(i, _):
                    v = rs_x[i] + rd_x[i]
                    l = jnp.where(v >= 0.0, v, 0.2 * v)
                    ex_x[i] = jnp.exp(l)
                    return 0

                lax.fori_loop(0, CHUNK, row_step, 0)
                pltpu.async_copy(ex_x, ex_hbm.at[c0 + c], sw_x)
                wr_idx(id_x, iad, c)
                pltpu.async_copy(ex_x, s_acc.at[id_x], sc_x, add=True)

                @pl.when(c + 2 < nch)
                def _():
                    wr_idx(is_x, ias, c + 2)
                    wr_idx(ig_x, iad, c + 2)
                    pltpu.async_copy(ts_hbm.at[is_x], rs_x, ss_x)
                    pltpu.async_copy(td_hbm.at[ig_x], rd_x, sd_x)
        return 0

    lax.fori_loop(0, (CPSB + 1) // 2, body, 0)
    for ex_x, sw_x, sc_x in ((ex_a, sw_a, sc_a), (ex_b, sw_b, sc_b)):
        pltpu.make_async_copy(ex_x, ex_hbm.at[0], sw_x).wait()
        pltpu.make_async_copy(ts_hbm.at[pl.ds(0, CHUNK)], ex_x,
                              sc_x).wait()
    plsc.subcore_barrier()
    pltpu.sync_copy(s_acc.at[pl.ds(r0, RPS)],
                    sp_hbm.at[core, pl.ds(r0, RPS)])


def _edge_phase(t_s, t_d, src3d, dst3d, zz16):
    mesh = plsc.VectorSubcoreMesh(core_axis_name="c", subcore_axis_name="s")
    return pl.kernel(
        _edge_body,
        compiler_params=pltpu.CompilerParams(use_tc_tiling_on_sc=False),
        out_type=[jax.ShapeDtypeStruct((NCHUNK, CHUNK, 2 * H), jnp.float32),
                  jax.ShapeDtypeStruct((NC, NPAD, 2 * H), jnp.float32)],
        mesh=mesh,
        scratch_types=[
            pltpu.VMEM((CPSB, 1, CHUNK), jnp.int32),
            pltpu.VMEM((CPSB, 1, CHUNK), jnp.int32),
            pltpu.VMEM((CHUNK,), jnp.int32),
            pltpu.VMEM((CHUNK,), jnp.int32),
            pltpu.VMEM((CHUNK,), jnp.int32),
            pltpu.VMEM((CHUNK,), jnp.int32),
            pltpu.VMEM((CHUNK,), jnp.int32),
            pltpu.VMEM((CHUNK,), jnp.int32),
            pltpu.VMEM((CHUNK, 2 * H), jnp.float32),
            pltpu.VMEM((CHUNK, 2 * H), jnp.float32),
            pltpu.VMEM((CHUNK, 2 * H), jnp.float32),
            pltpu.VMEM((CHUNK, 2 * H), jnp.float32),
            pltpu.VMEM((CHUNK, 2 * H), jnp.float32),
            pltpu.VMEM((CHUNK, 2 * H), jnp.float32),
            pltpu.SemaphoreType.DMA,
            pltpu.SemaphoreType.DMA,
            pltpu.SemaphoreType.DMA,
            pltpu.SemaphoreType.DMA,
            pltpu.SemaphoreType.DMA,
            pltpu.SemaphoreType.DMA,
            pltpu.SemaphoreType.DMA,
            pltpu.SemaphoreType.DMA,
            pltpu.VMEM_SHARED((NPAD, 2 * H), jnp.float32),
        ],
    )(t_s, t_d, src3d, dst3d, zz16)


CH = 64
NCHC = E // CH
CPSC = (NCHC + NS - 1) // NS
HC = 80
NQ = 4
NCHCP = NS * CPSC + NQ * HC


def _agg_body(ndc, xflat_hbm, src_hbm, dst_hbm, ex_hbm, zz_hbm, u_hbm,
              ias, iad, is_a, is_b, id_a, id_b, x_a, x_b, ex_a, ex_b,
              sr_a, sr_b, sg_a, sg_b, sx_a, sx_b, sc_a, sc_b, u_acc):
    pairs_per_sc = (H * ndc) // NC
    core = lax.axis_index("c")
    sub = lax.axis_index("s")
    r0 = sub * RPS
    c0 = sub * CPSC
    nch = jnp.minimum(CPSC, NCHC - c0)
    dnums = lax.GatherDimensionNumbers(
        offset_dims=(), collapsed_slice_dims=(0,), start_index_map=(0,))

    def wr_idx(dref, sref, c, off):
        for q in range(CH // 16):
            sl = pl.ds(q * 16, 16)
            dref[sl] = sref[c, 0, sl] + off

    def pair_step(p_idx, _):
        pid = core * pairs_per_sc + p_idx
        h = pid // ndc
        dc = pid % ndc
        off = dc * NPAD
        hsplat = jnp.zeros((16,), jnp.int32) + h

        pltpu.sync_copy(zz_hbm.at[pl.ds(r0, RPS)], u_acc.at[pl.ds(r0, RPS)])
        plsc.subcore_barrier()

        for half in range(NQ):
            h0 = half * HC
            lim = jnp.clip(nch - h0, 0, HC)
            pltpu.sync_copy(src_hbm.at[pl.ds(c0 + h0, HC)], ias)
            pltpu.sync_copy(dst_hbm.at[pl.ds(c0 + h0, HC)], iad)

            wr_idx(is_a, ias, 0, off)
            pltpu.async_copy(xflat_hbm.at[is_a], x_a, sg_a)
            pltpu.async_copy(ex_hbm.at[c0 + h0], ex_a, sx_a)
            wr_idx(is_b, ias, 1, off)
            pltpu.async_copy(xflat_hbm.at[is_b], x_b, sg_b)
            pltpu.async_copy(ex_hbm.at[c0 + h0 + 1], ex_b, sx_b)

            def body(j, _):
                for par, is_x, id_x, x_x, ex_x, sr_x, sg_x, sx_x, sc_x in (
                        (0, is_a, id_a, x_a, ex_a, sr_a, sg_a, sx_a, sc_a),
                        (1, is_b, id_b, x_b, ex_b, sr_b, sg_b, sx_b, sc_b)):
                    lc = 2 * j + par

                    @pl.when(lc < lim)
                    def _():
                        pltpu.make_async_copy(
                            xflat_hbm.at[pl.ds(0, CH)], x_x, sg_x).wait()
                        pltpu.make_async_copy(
                            ex_hbm.at[0], ex_x, sx_x).wait()

                        @pl.when(j > 0)
                        def _():
                            pltpu.make_async_copy(
                                xflat_hbm.at[pl.ds(0, CH)], sr_x,
                                sc_x).wait()

                        def row_step(i, _):
                            a = lax.gather(
                                ex_x[i], hsplat[:, None], dnums, (1,),
                                mode=lax.GatherScatterMode.PROMISE_IN_BOUNDS)
                            for q in range(8):
                                sl = pl.ds(q * 16, 16)
                                sr_x[i, sl] = x_x[i, sl] * a
                            return 0

                        lax.fori_loop(0, CH, row_step, 0)
                        wr_idx(id_x, iad, lc, 0)
                        pltpu.async_copy(sr_x, u_acc.at[id_x], sc_x,
                                         add=True)

                        @pl.when(lc + 2 < lim)
                        def _():
                            wr_idx(is_x, ias, lc + 2, off)
                            pltpu.async_copy(xflat_hbm.at[is_x], x_x, sg_x)
                            pltpu.async_copy(ex_hbm.at[c0 + h0 + lc + 2],
                                             ex_x, sx_x)
                return 0

            lax.fori_loop(0, HC // 2, body, 0)
            pltpu.make_async_copy(xflat_hbm.at[pl.ds(0, CH)], sr_a,
                                  sc_a).wait()
            pltpu.make_async_copy(xflat_hbm.at[pl.ds(0, CH)], sr_b,
                                  sc_b).wait()
        plsc.subcore_barrier()
        pltpu.sync_copy(u_acc.at[pl.ds(r0, RPS)],
                        u_hbm.at[pid, pl.ds(r0, RPS)])
        return 0

    lax.fori_loop(0, pairs_per_sc, pair_step, 0)


def _aggregate(xflat, src16c, dst16c, ex_c, zz128, ndc):
    mesh = plsc.VectorSubcoreMesh(core_axis_name="c", subcore_axis_name="s")
    return pl.kernel(
        functools.partial(_agg_body, ndc),
        out_type=jax.ShapeDtypeStruct((H * ndc, NPAD, 128), jnp.float32),
        mesh=mesh,
        compiler_params=pltpu.CompilerParams(use_tc_tiling_on_sc=False),
        scratch_types=[
            pltpu.VMEM((HC, 1, CH), jnp.int32),
            pltpu.VMEM((HC, 1, CH), jnp.int32),
            pltpu.VMEM((CH,), jnp.int32),
            pltpu.VMEM((CH,), jnp.int32),
            pltpu.VMEM((CH,), jnp.int32),
            pltpu.VMEM((CH,), jnp.int32),
            pltpu.VMEM((CH, 128), jnp.float32),
            pltpu.VMEM((CH, 128), jnp.float32),
            pltpu.VMEM((CH, 2 * H), jnp.float32),
            pltpu.VMEM((CH, 2 * H), jnp.float32),
            pltpu.VMEM((CH, 128), jnp.float32),
            pltpu.VMEM((CH, 128), jnp.float32),
            pltpu.SemaphoreType.DMA,
            pltpu.SemaphoreType.DMA,
            pltpu.SemaphoreType.DMA,
            pltpu.SemaphoreType.DMA,
            pltpu.SemaphoreType.DMA,
            pltpu.SemaphoreType.DMA,
            pltpu.VMEM_SHARED((NPAD, 128), jnp.float32),
        ],
    )(xflat, src16c, dst16c, ex_c, zz128)


def _comb_body(ndc, ndc_out, final, u_ref, sp_ref, wt_ref, b_ref,
               wlin_ref, blin_ref, o_ref):
    pairs = H * ndc
    s = sp_ref[0, :, :H] + sp_ref[1, :, :H] + 1e-16
    acc = jnp.zeros((u_ref.shape[1], C), jnp.float32)
    for pid in range(pairs):
        h = pid // ndc
        r = u_ref[pid] / s[:, h][:, None]
        acc = acc + jnp.dot(r, wt_ref[pid],
                            preferred_element_type=jnp.float32)
    y = jax.nn.relu(acc * (1.0 / H) + b_ref[...][None])
    if final:
        o_ref[...] = jnp.dot(y, wlin_ref[...],
                             preferred_element_type=jnp.float32) \
            + blin_ref[...][None]
    else:
        for j in range(ndc_out):
            o_ref[j] = y[:, j * 128:(j + 1) * 128]


def _combine(u, s_part, W_t, b, W_lin, b_lin, ndc, final):
    bn = 1024
    pairs = H * ndc
    ndc_out = C // 128
    if final:
        out_shape = jax.ShapeDtypeStruct((NPAD, OUT), jnp.float32)
        out_spec = pl.BlockSpec((bn, OUT), lambda i: (i, 0))
    else:
        out_shape = jax.ShapeDtypeStruct((ndc_out, NPAD, 128), jnp.float32)
        out_spec = pl.BlockSpec((ndc_out, bn, 128), lambda i: (0, i, 0))
    return pl.pallas_call(
        functools.partial(_comb_body, ndc, ndc_out, final),
        grid=(NPAD // bn,),
        in_specs=[
            pl.BlockSpec((pairs, bn, 128), lambda i: (0, i, 0)),
            pl.BlockSpec((NC, bn, 2 * H), lambda i: (0, i, 0)),
            pl.BlockSpec((pairs, 128, C), lambda i: (0, 0, 0)),
            pl.BlockSpec((C,), lambda i: (0,)),
            pl.BlockSpec((C, OUT), lambda i: (0, 0)),
            pl.BlockSpec((OUT,), lambda i: (0,)),
        ],
        out_specs=out_spec,
        out_shape=out_shape,
    )(u, s_part, W_t, b, W_lin, b_lin)


def kernel(x, edge_index, W0, a_src0, a_dst0, b0, W1, a_src1, a_dst1, b1,
           W2, a_src2, a_dst2, b2, W_lin, b_lin):
    src32 = edge_index[0].astype(jnp.int32)
    dst32 = edge_index[1].astype(jnp.int32)
    src16b = jnp.pad(src32.reshape(NCHUNK, 1, CHUNK),
                     ((0, NCHBP - NCHUNK), (0, 0), (0, 0)))
    dst16b = jnp.pad(dst32.reshape(NCHUNK, 1, CHUNK),
                     ((0, NCHBP - NCHUNK), (0, 0), (0, 0)))
    src16c = jnp.pad(src32.reshape(NCHC, 1, CH),
                     ((0, NCHCP - NCHC), (0, 0), (0, 0)))
    dst16c = jnp.pad(dst32.reshape(NCHC, 1, CH),
                     ((0, NCHCP - NCHC), (0, 0), (0, 0)))
    zz16 = jnp.zeros((NPAD, 2 * H), jnp.float32)
    zz128 = jnp.zeros((NPAD, 128), jnp.float32)

    x_cur = jnp.pad(x, ((0, NPAD - N), (0, 0))).reshape(1, NPAD, 128)
    layers = ((W0, a_src0, a_dst0, b0), (W1, a_src1, a_dst1, b1),
              (W2, a_src2, a_dst2, b2))
    for l, (W, a_s, a_d, b) in enumerate(layers):
        ndc = x_cur.shape[0]
        t_s, t_d = _proj(x_cur, W, a_s, a_d)
        ex_e, s_part = _edge_phase(t_s, t_d, src16b, dst16b, zz16)
        xflat = x_cur.reshape(ndc * NPAD, 128)
        ex_c = ex_e.reshape(NCHC, CH, 2 * H)
        u = _aggregate(xflat, src16c, dst16c, ex_c, zz128, ndc)
        W_t = W.reshape(ndc, 128, H, C).transpose(2, 0, 1, 3) \
            .reshape(H * ndc, 128, C)
        x_cur = _combine(u, s_part, W_t, b, W_lin, b_lin, ndc,
                         final=(l == 2))
    return x_cur[:N]


# speedup vs baseline: 21.4306x; 1.0302x over previous
import functools
import jax
import jax.numpy as jnp
from jax import lax
from jax.experimental import pallas as pl
from jax.experimental.pallas import tpu as pltpu
import jax.experimental.pallas.tpu_sc as plsc

N = 10000
NPAD = 10240
E = 320000
H = 8
C = 256
OUT = 10

CHUNK = 128
NCHUNK = E // CHUNK
NC, NS = 2, 16
NW = NC * NS
RPS = NPAD // NS


def _proj_body(ndc, x_ref, w_ref, as_ref, ad_ref, ts_ref, td_ref):
    xs = jnp.concatenate([x_ref[i] for i in range(ndc)], axis=1)
    w = w_ref[...]
    ws = jnp.sum(w * as_ref[...][None], axis=2)
    wd = jnp.sum(w * ad_ref[...][None], axis=2)
    es = jnp.dot(xs, ws, preferred_element_type=jnp.float32)
    ed = jnp.dot(xs, wd, preferred_element_type=jnp.float32)
    ts_ref[...] = jnp.concatenate([es, es], axis=1)
    td_ref[...] = jnp.concatenate([ed, ed], axis=1)


def _proj(x4, W, a_s, a_d):
    ndc = x4.shape[0]
    return pl.pallas_call(
        functools.partial(_proj_body, ndc),
        out_shape=[jax.ShapeDtypeStruct((NPAD, 2 * H), jnp.float32),
                   jax.ShapeDtypeStruct((NPAD, 2 * H), jnp.float32)],
    )(x4, W, a_s, a_d)


CPSB = (NCHUNK + NW - 1) // NW
NCHBP = NW * CPSB


def _edge_body(ts_hbm, td_hbm, src_hbm, dst_hbm, zz_hbm, ex_hbm, sp_hbm,
               ias, iad, is_a, is_b, ig_a, ig_b, id_a, id_b,
               rs_a, rs_b, rd_a, rd_b, ex_a, ex_b,
               ss_a, ss_b, sd_a, sd_b, sw_a, sw_b, sc_a, sc_b, s_acc):
    core = lax.axis_index("c")
    sub = lax.axis_index("s")
    r0 = sub * RPS
    c0 = (sub * NC + core) * CPSB
    nch = jnp.minimum(CPSB, NCHUNK - c0)

    pltpu.sync_copy(zz_hbm.at[pl.ds(r0, RPS)], s_acc.at[pl.ds(r0, RPS)])
    pltpu.sync_copy(src_hbm.at[pl.ds(c0, CPSB)], ias)
    pltpu.sync_copy(dst_hbm.at[pl.ds(c0, CPSB)], iad)
    plsc.subcore_barrier()

    def wr_idx(dref, sref, c):
        for q in range(CHUNK // 16):
            sl = pl.ds(q * 16, 16)
            dref[sl] = sref[c, 0, sl]

    wr_idx(is_a, ias, 0)
    wr_idx(ig_a, iad, 0)
    pltpu.async_copy(ts_hbm.at[is_a], rs_a, ss_a)
    pltpu.async_copy(td_hbm.at[ig_a], rd_a, sd_a)
    wr_idx(is_b, ias, 1)
    wr_idx(ig_b, iad, 1)
    pltpu.async_copy(ts_hbm.at[is_b], rs_b, ss_b)
    pltpu.async_copy(td_hbm.at[ig_b], rd_b, sd_b)

    def body(j, _):
        for par, is_x, ig_x, id_x, rs_x, rd_x, ex_x, ss_x, sd_x, sw_x, \
                sc_x in (
                (0, is_a, ig_a, id_a, rs_a, rd_a, ex_a, ss_a, sd_a, sw_a,
                 sc_a),
                (1, is_b, ig_b, id_b, rs_b, rd_b, ex_b, ss_b, sd_b, sw_b,
                 sc_b)):
            c = 2 * j + par

            @pl.when(c < nch)
            def _():
                pltpu.make_async_copy(ts_hbm.at[pl.ds(0, CHUNK)], rs_x,
                                      ss_x).wait()
                pltpu.make_async_copy(td_hbm.at[pl.ds(0, CHUNK)], rd_x,
                                      sd_x).wait()

                @pl.when(j > 0)
                def _():
                    pltpu.make_async_copy(ex_x, ex_hbm.at[0], sw_x).wait()
                    pltpu.make_async_copy(ts_hbm.at[pl.ds(0, CHUNK)],
                                          ex_x, sc_x).wait()

                def row_step(i, _):
                    v = rs_x[i] + rd_x[i]
                    l = jnp.where(v >= 0.0, v, 0.2 * v)
                    ex_x[i] = jnp.exp(l)
                    return 0

                lax.fori_loop(0, CHUNK, row_step, 0)
                pltpu.async_copy(ex_x, ex_hbm.at[c0 + c], sw_x)
                wr_idx(id_x, iad, c)
                pltpu.async_copy(ex_x, s_acc.at[id_x], sc_x, add=True)

                @pl.when(c + 2 < nch)
                def _():
                    wr_idx(is_x, ias, c + 2)
                    wr_idx(ig_x, iad, c + 2)
                    pltpu.async_copy(ts_hbm.at[is_x], rs_x, ss_x)
                    pltpu.async_copy(td_hbm.at[ig_x], rd_x, sd_x)
        return 0

    lax.fori_loop(0, (CPSB + 1) // 2, body, 0)
    for ex_x, sw_x, sc_x in ((ex_a, sw_a, sc_a), (ex_b, sw_b, sc_b)):
        pltpu.make_async_copy(ex_x, ex_hbm.at[0], sw_x).wait()
        pltpu.make_async_copy(ts_hbm.at[pl.ds(0, CHUNK)], ex_x,
                              sc_x).wait()
    plsc.subcore_barrier()
    pltpu.sync_copy(s_acc.at[pl.ds(r0, RPS)],
                    sp_hbm.at[core, pl.ds(r0, RPS)])


def _edge_phase(t_s, t_d, src3d, dst3d, zz16):
    mesh = plsc.VectorSubcoreMesh(core_axis_name="c", subcore_axis_name="s")
    return pl.kernel(
        _edge_body,
        compiler_params=pltpu.CompilerParams(use_tc_tiling_on_sc=False),
        out_type=[jax.ShapeDtypeStruct((NCHUNK, CHUNK, 2 * H), jnp.float32),
                  jax.ShapeDtypeStruct((NC, NPAD, 2 * H), jnp.float32)],
        mesh=mesh,
        scratch_types=[
            pltpu.VMEM((CPSB, 1, CHUNK), jnp.int32),
            pltpu.VMEM((CPSB, 1, CHUNK), jnp.int32),
            pltpu.VMEM((CHUNK,), jnp.int32),
            pltpu.VMEM((CHUNK,), jnp.int32),
            pltpu.VMEM((CHUNK,), jnp.int32),
            pltpu.VMEM((CHUNK,), jnp.int32),
            pltpu.VMEM((CHUNK,), jnp.int32),
            pltpu.VMEM((CHUNK,), jnp.int32),
            pltpu.VMEM((CHUNK, 2 * H), jnp.float32),
            pltpu.VMEM((CHUNK, 2 * H), jnp.float32),
            pltpu.VMEM((CHUNK, 2 * H), jnp.float32),
            pltpu.VMEM((CHUNK, 2 * H), jnp.float32),
            pltpu.VMEM((CHUNK, 2 * H), jnp.float32),
            pltpu.VMEM((CHUNK, 2 * H), jnp.float32),
            pltpu.SemaphoreType.DMA,
            pltpu.SemaphoreType.DMA,
            pltpu.SemaphoreType.DMA,
            pltpu.SemaphoreType.DMA,
            pltpu.SemaphoreType.DMA,
            pltpu.SemaphoreType.DMA,
            pltpu.SemaphoreType.DMA,
            pltpu.SemaphoreType.DMA,
            pltpu.VMEM_SHARED((NPAD, 2 * H), jnp.float32),
        ],
    )(t_s, t_d, src3d, dst3d, zz16)


CH = 64
NCHC = E // CH
CPSC = (NCHC + NS - 1) // NS
HC = 80
NQ = 4
NCHCP = NS * CPSC + NQ * HC


def _agg_body(ndc, xflat_hbm, src_hbm, dst_hbm, ex_hbm, zz_hbm, u_hbm,
              ias, iad, is_a, is_b, id_a, id_b, x_a, x_b, ex_a, ex_b,
              sr_a, sr_b, sg_a, sg_b, sx_a, sx_b, sc_a, sc_b, u_acc):
    pairs_per_sc = (H * ndc) // NC
    core = lax.axis_index("c")
    sub = lax.axis_index("s")
    r0 = sub * RPS
    c0 = sub * CPSC
    nch = jnp.minimum(CPSC, NCHC - c0)
    dnums = lax.GatherDimensionNumbers(
        offset_dims=(), collapsed_slice_dims=(0,), start_index_map=(0,))

    def wr_idx(dref, sref, c, off):
        for q in range(CH // 16):
            sl = pl.ds(q * 16, 16)
            dref[sl] = sref[c, 0, sl] + off

    def pair_step(p_idx, _):
        pid = core * pairs_per_sc + p_idx
        h = pid // ndc
        dc = pid % ndc
        off = dc * NPAD
        hsplat = jnp.zeros((16,), jnp.int32) + h

        pltpu.sync_copy(zz_hbm.at[pl.ds(r0, RPS)], u_acc.at[pl.ds(r0, RPS)])
        plsc.subcore_barrier()

        for half in range(NQ):
            h0 = half * HC
            lim = jnp.clip(nch - h0, 0, HC)
            pltpu.sync_copy(src_hbm.at[pl.ds(c0 + h0, HC)], ias)
            pltpu.sync_copy(dst_hbm.at[pl.ds(c0 + h0, HC)], iad)

            wr_idx(is_a, ias, 0, off)
            pltpu.async_copy(xflat_hbm.at[is_a], x_a, sg_a)
            pltpu.async_copy(ex_hbm.at[c0 + h0], ex_a, sx_a)
            wr_idx(is_b, ias, 1, off)
            pltpu.async_copy(xflat_hbm.at[is_b], x_b, sg_b)
            pltpu.async_copy(ex_hbm.at[c0 + h0 + 1], ex_b, sx_b)

            def body(j, _):
                for par, is_x, id_x, x_x, ex_x, sr_x, sg_x, sx_x, sc_x in (
                        (0, is_a, id_a, x_a, ex_a, sr_a, sg_a, sx_a, sc_a),
                        (1, is_b, id_b, x_b, ex_b, sr_b, sg_b, sx_b, sc_b)):
                    lc = 2 * j + par

                    @pl.when(lc < lim)
                    def _():
                        pltpu.make_async_copy(
                            xflat_hbm.at[pl.ds(0, CH)], x_x, sg_x).wait()
                        pltpu.make_async_copy(
                            ex_hbm.at[0], ex_x, sx_x).wait()

                        @pl.when(j > 0)
                        def _():
                            pltpu.make_async_copy(
                                xflat_hbm.at[pl.ds(0, CH)], sr_x,
                                sc_x).wait()

                        def row_step(i, _):
                            a = lax.gather(
                                ex_x[i], hsplat[:, None], dnums, (1,),
                                mode=lax.GatherScatterMode.PROMISE_IN_BOUNDS)
                            for q in range(8):
                                sl = pl.ds(q * 16, 16)
                                sr_x[i, sl] = x_x[i, sl] * a
                            return 0

                        lax.fori_loop(0, CH, row_step, 0)
                        wr_idx(id_x, iad, lc, 0)
                        pltpu.async_copy(sr_x, u_acc.at[id_x], sc_x,
                                         add=True)

                        @pl.when(lc + 2 < lim)
                        def _():
                            wr_idx(is_x, ias, lc + 2, off)
                            pltpu.async_copy(xflat_hbm.at[is_x], x_x, sg_x)
                            pltpu.async_copy(ex_hbm.at[c0 + h0 + lc + 2],
                                             ex_x, sx_x)
                return 0

            lax.fori_loop(0, HC // 2, body, 0)
            pltpu.make_async_copy(xflat_hbm.at[pl.ds(0, CH)], sr_a,
                                  sc_a).wait()
            pltpu.make_async_copy(xflat_hbm.at[pl.ds(0, CH)], sr_b,
                                  sc_b).wait()
        plsc.subcore_barrier()
        pltpu.sync_copy(u_acc.at[pl.ds(r0, RPS)],
                        u_hbm.at[pid, pl.ds(r0, RPS)])
        return 0

    lax.fori_loop(0, pairs_per_sc, pair_step, 0)


def _aggregate(xflat, src16c, dst16c, ex_c, zz128, ndc):
    mesh = plsc.VectorSubcoreMesh(core_axis_name="c", subcore_axis_name="s")
    return pl.kernel(
        functools.partial(_agg_body, ndc),
        out_type=jax.ShapeDtypeStruct((H * ndc, NPAD, 128), jnp.float32),
        mesh=mesh,
        compiler_params=pltpu.CompilerParams(use_tc_tiling_on_sc=False),
        scratch_types=[
            pltpu.VMEM((HC, 1, CH), jnp.int32),
            pltpu.VMEM((HC, 1, CH), jnp.int32),
            pltpu.VMEM((CH,), jnp.int32),
            pltpu.VMEM((CH,), jnp.int32),
            pltpu.VMEM((CH,), jnp.int32),
            pltpu.VMEM((CH,), jnp.int32),
            pltpu.VMEM((CH, 128), jnp.float32),
            pltpu.VMEM((CH, 128), jnp.float32),
            pltpu.VMEM((CH, 2 * H), jnp.float32),
            pltpu.VMEM((CH, 2 * H), jnp.float32),
            pltpu.VMEM((CH, 128), jnp.float32),
            pltpu.VMEM((CH, 128), jnp.float32),
            pltpu.SemaphoreType.DMA,
            pltpu.SemaphoreType.DMA,
            pltpu.SemaphoreType.DMA,
            pltpu.SemaphoreType.DMA,
            pltpu.SemaphoreType.DMA,
            pltpu.SemaphoreType.DMA,
            pltpu.VMEM_SHARED((NPAD, 128), jnp.float32),
        ],
    )(xflat, src16c, dst16c, ex_c, zz128)


def _comb_body(ndc, ndc_out, final, u_ref, sp_ref, wt_ref, b_ref,
               wlin_ref, blin_ref, o_ref):
    pairs = H * ndc
    s = sp_ref[0, :, :H] + sp_ref[1, :, :H] + 1e-16
    acc = jnp.zeros((u_ref.shape[1], C), jnp.float32)
    for pid in range(pairs):
        h = pid // ndc
        r = u_ref[pid] / s[:, h][:, None]
        acc = acc + jnp.dot(r, wt_ref[pid],
                            preferred_element_type=jnp.float32)
    y = jax.nn.relu(acc * (1.0 / H) + b_ref[...][None])
    if final:
        o_ref[...] = jnp.dot(y, wlin_ref[...],
                             preferred_element_type=jnp.float32) \
            + blin_ref[...][None]
    else:
        for j in range(ndc_out):
            o_ref[j] = y[:, j * 128:(j + 1) * 128]


def _combine(u, s_part, W_t, b, W_lin, b_lin, ndc, final):
    bn = 1024
    pairs = H * ndc
    ndc_out = C // 128
    if final:
        out_shape = jax.ShapeDtypeStruct((NPAD, OUT), jnp.float32)
        out_spec = pl.BlockSpec((bn, OUT), lambda i: (i, 0))
    else:
        out_shape = jax.ShapeDtypeStruct((ndc_out, NPAD, 128), jnp.float32)
        out_spec = pl.BlockSpec((ndc_out, bn, 128), lambda i: (0, i, 0))
    return pl.pallas_call(
        functools.partial(_comb_body, ndc, ndc_out, final),
        grid=(NPAD // bn,),
        in_specs=[
            pl.BlockSpec((pairs, bn, 128), lambda i: (0, i, 0)),
            pl.BlockSpec((NC, bn, 2 * H), lambda i: (0, i, 0)),
            pl.BlockSpec((pairs, 128, C), lambda i: (0, 0, 0)),
            pl.BlockSpec((C,), lambda i: (0,)),
            pl.BlockSpec((C, OUT), lambda i: (0, 0)),
            pl.BlockSpec((OUT,), lambda i: (0,)),
        ],
        out_specs=out_spec,
        out_shape=out_shape,
    )(u, s_part, W_t, b, W_lin, b_lin)


def kernel(x, edge_index, W0, a_src0, a_dst0, b0, W1, a_src1, a_dst1, b1,
           W2, a_src2, a_dst2, b2, W_lin, b_lin):
    src32 = edge_index[0].astype(jnp.int32)
    dst32 = edge_index[1].astype(jnp.int32)
    src16b = jnp.pad(src32.reshape(NCHUNK, 1, CHUNK),
                     ((0, NCHBP - NCHUNK), (0, 0), (0, 0)))
    dst16b = jnp.pad(dst32.reshape(NCHUNK, 1, CHUNK),
                     ((0, NCHBP - NCHUNK), (0, 0), (0, 0)))
    src16c = jnp.pad(src32.reshape(NCHC, 1, CH),
                     ((0, NCHCP - NCHC), (0, 0), (0, 0)))
    dst16c = jnp.pad(dst32.reshape(NCHC, 1, CH),
                     ((0, NCHCP - NCHC), (0, 0), (0, 0)))
    zz16 = jnp.zeros((NPAD, 2 * H), jnp.float32)
    zz128 = jnp.zeros((NPAD, 128), jnp.float32)

    x_cur = jnp.pad(x, ((0, NPAD - N), (0, 0))).reshape(1, NPAD, 128)
    layers = ((W0, a_src0, a_dst0, b0), (W1, a_src1, a_dst1, b1),
              (W2, a_src2, a_dst2, b2))
    for l, (W, a_s, a_d, b) in enumerate(layers):
        ndc = x_cur.shape[0]
        t_s, t_d = _proj(x_cur, W, a_s, a_d)
        ex_e, s_part = _edge_phase(t_s, t_d, src16b, dst16b, zz16)
        xflat = x_cur.reshape(ndc * NPAD, 128)
        ex_c = ex_e.reshape(NCHC, CH, 2 * H)
        u = _aggregate(xflat, src16c, dst16c, ex_c, zz128, ndc)
        W_t = W.reshape(ndc, 128, H, C).transpose(2, 0, 1, 3) \
            .reshape(H * ndc, 128, C)
        x_cur = _combine(u, s_part, W_t, b, W_lin, b_lin, ndc,
                         final=(l == 2))
    return x_cur[:N]
